# baseline (speedup 1.0000x reference)
import jax
import jax.numpy as jnp
from jax.experimental import pallas as pl
from jax.experimental.pallas import tpu as pltpu

_ALPHA = 4
_FPB = 16


def kernel(frames):
    C, T, H, W = frames.shape
    n_slow = T // _ALPHA
    spb = _FPB // _ALPHA
    idx = jnp.linspace(0.0, T - 1, n_slow).astype(jnp.int32)
    off = idx - _ALPHA * jnp.arange(n_slow, dtype=jnp.int32)

    def body(off_ref, in_ref, slow_ref, fast_ref):
        fast_ref[...] = in_ref[...]
        g = pl.program_id(0)
        for s in range(spb):
            o = off_ref[g * spb + s] + s * _ALPHA
            slow_ref[:, pl.ds(s, 1)] = in_ref[:, pl.ds(o, 1)]

    grid_spec = pltpu.PrefetchScalarGridSpec(
        num_scalar_prefetch=1,
        grid=(T // _FPB,),
        in_specs=[
            pl.BlockSpec((C, _FPB, H, W), lambda g, off_r: (0, g, 0, 0)),
        ],
        out_specs=[
            pl.BlockSpec((C, spb, H, W), lambda g, off_r: (0, g, 0, 0)),
            pl.BlockSpec((C, _FPB, H, W), lambda g, off_r: (0, g, 0, 0)),
        ],
    )
    slow, fast = pl.pallas_call(
        body,
        grid_spec=grid_spec,
        out_shape=(
            jax.ShapeDtypeStruct((C, n_slow, H, W), frames.dtype),
            jax.ShapeDtypeStruct((C, T, H, W), frames.dtype),
        ),
        compiler_params=pltpu.CompilerParams(
            dimension_semantics=("parallel",),
        ),
    )(off, frames)
    return (slow, fast)
